# Initial kernel scaffold; baseline (speedup 1.0000x reference)
import jax
import jax.numpy as jnp
from jax.experimental import pallas as pl


def kernel(pos, batch, weight):
    raise NotImplementedError("write your pallas kernel here")


# speedup vs baseline: 3.5480x; 3.5480x over previous
import jax
import jax.numpy as jnp
from jax import lax
from jax.experimental import pallas as pl
from jax.experimental.pallas import tpu as pltpu
from jax.experimental.pallas import tpu_sc as plsc

N = 100000
NUM_GRAPHS = 256
EPS = 1e-05

L = 16
NC = 2
NS = 16
NW = NC * NS
PW = 3136
NPAD = NW * PW
BINS = 272
KV = PW // L

_MESH = plsc.VectorSubcoreMesh(core_axis_name="c", subcore_axis_name="s",
                               num_cores=NC, num_subcores=NS)
_PARAMS = pltpu.CompilerParams(needs_layout_passes=False)


def _nsqrt(q):
    i = plsc.bitcast(q, jnp.int32)
    g = plsc.bitcast((i >> 1) + 0x1FBD1DF5, jnp.float32)
    for _ in range(3):
        g = 0.5 * (g + q / g)
    return g


def _wid():
    return lax.axis_index("s") * NC + lax.axis_index("c")


def _stats_body(pos_hbm, batch_hbm, part_hbm, posb, bb, psum, pcnt, locb):
    wid = _wid()
    iota = lax.iota(jnp.int32, L)
    zeros = jnp.zeros((L,), jnp.float32)
    ones = jnp.full((L,), 1.0, jnp.float32)

    pltpu.sync_copy(pos_hbm.at[pl.ds(wid * (3 * PW), 3 * PW)], posb)
    pltpu.sync_copy(batch_hbm.at[pl.ds(wid * PW, PW)], bb)

    def zero_body(v, _):
        for l in range(L):
            psum[l, pl.ds(v * L, L)] = zeros
            pcnt[l, pl.ds(v * L, L)] = zeros
        return 0
    lax.fori_loop(0, BINS // L, zero_body, 0)

    def pass1(k, _):
        base = k * (3 * L) + iota * 3
        x = plsc.load_gather(posb, [base])
        y = plsc.load_gather(posb, [base + 1])
        z = plsc.load_gather(posb, [base + 2])
        nrm = _nsqrt(x * x + y * y + z * z)
        b = bb[pl.ds(k * L, L)]
        plsc.addupdate_scatter(psum, [iota, b], nrm)
        plsc.addupdate_scatter(pcnt, [iota, b], ones)
        return 0
    lax.fori_loop(0, KV, pass1, 0)

    def lred(v, _):
        s = psum[0, pl.ds(v * L, L)]
        c = pcnt[0, pl.ds(v * L, L)]
        for l in range(1, L):
            s = s + psum[l, pl.ds(v * L, L)]
            c = c + pcnt[l, pl.ds(v * L, L)]
        locb[pl.ds(v * L, L)] = s
        locb[pl.ds(BINS + v * L, L)] = c
        return 0
    lax.fori_loop(0, BINS // L, lred, 0)

    pltpu.sync_copy(locb, part_hbm.at[wid])


def _norm_body(pos_hbm, batch_hbm, w_hbm, part_hbm, out_hbm,
               posb, outb, bb, partb, scaleb, wv):
    wid = _wid()
    iota = lax.iota(jnp.int32, L)

    pltpu.sync_copy(pos_hbm.at[pl.ds(wid * (3 * PW), 3 * PW)], posb)
    pltpu.sync_copy(batch_hbm.at[pl.ds(wid * PW, PW)], bb)
    pltpu.sync_copy(part_hbm, partb)
    pltpu.sync_copy(w_hbm, wv)
    wvec = wv[...]

    def gred(v, _):
        s = partb[0, pl.ds(v * L, L)]
        c = partb[0, pl.ds(BINS + v * L, L)]
        for t in range(1, NW):
            s = s + partb[t, pl.ds(v * L, L)]
            c = c + partb[t, pl.ds(BINS + v * L, L)]
        mean = s / jnp.maximum(c, 1.0)
        scaleb[pl.ds(v * L, L)] = wvec / (mean + EPS)
        return 0
    lax.fori_loop(0, BINS // L, gred, 0)

    def pass2(k, _):
        b = bb[pl.ds(k * L, L)]
        sc = plsc.load_gather(scaleb, [b])
        base = k * (3 * L) + iota * 3
        for c in range(3):
            v = plsc.load_gather(posb, [base + c])
            plsc.store_scatter(outb, [base + c], v * sc)
        return 0
    lax.fori_loop(0, KV, pass2, 0)

    pltpu.sync_copy(outb, out_hbm.at[pl.ds(wid * (3 * PW), 3 * PW)])


_stats_sc = pl.kernel(
    _stats_body,
    out_type=jax.ShapeDtypeStruct((NW, 2 * BINS), jnp.float32),
    mesh=_MESH,
    compiler_params=_PARAMS,
    scratch_types=[
        pltpu.VMEM((3 * PW,), jnp.float32),
        pltpu.VMEM((PW,), jnp.int32),
        pltpu.VMEM((L, BINS), jnp.float32),
        pltpu.VMEM((L, BINS), jnp.float32),
        pltpu.VMEM((2 * BINS,), jnp.float32),
    ],
)

_norm_sc = pl.kernel(
    _norm_body,
    out_type=jax.ShapeDtypeStruct((3 * NPAD,), jnp.float32),
    mesh=_MESH,
    compiler_params=_PARAMS,
    scratch_types=[
        pltpu.VMEM((3 * PW,), jnp.float32),
        pltpu.VMEM((3 * PW,), jnp.float32),
        pltpu.VMEM((PW,), jnp.int32),
        pltpu.VMEM((NW, 2 * BINS), jnp.float32),
        pltpu.VMEM((BINS,), jnp.float32),
        pltpu.VMEM((L,), jnp.float32),
    ],
)


def kernel(pos, batch, weight):
    pos_pad = jnp.pad(pos.astype(jnp.float32), ((0, NPAD - N), (0, 0)))
    batch_pad = jnp.pad(batch.astype(jnp.int32), (0, NPAD - N),
                        constant_values=NUM_GRAPHS)
    wvec = jnp.full((L,), 1.0, jnp.float32) * weight[0, 0]
    posf = pos_pad.reshape(-1)
    part = _stats_sc(posf, batch_pad)
    outf = _norm_sc(posf, batch_pad, wvec, part)
    return outf.reshape(NPAD, 3)[:N]


# speedup vs baseline: 15.3983x; 4.3401x over previous
import jax
import jax.numpy as jnp
from jax import lax
from jax.experimental import pallas as pl
from jax.experimental.pallas import tpu as pltpu
from jax.experimental.pallas import tpu_sc as plsc

N = 100000
NUM_GRAPHS = 256
EPS = 1e-05

L = 16
NC = 2
NS = 16
NW = NC * NS
PW = 3136
NPAD = NW * PW
BINS = 272
KV = PW // L

_MESH = plsc.VectorSubcoreMesh(core_axis_name="c", subcore_axis_name="s",
                               num_cores=NC, num_subcores=NS)
_PARAMS = pltpu.CompilerParams(needs_layout_passes=False)


def _nsqrt(q):
    i = plsc.bitcast(q, jnp.int32)
    g = plsc.bitcast((i >> 1) + 0x1FBD1DF5, jnp.float32)
    for _ in range(3):
        g = 0.5 * (g + q / g)
    return g


def _wid():
    return lax.axis_index("s") * NC + lax.axis_index("c")


def _stats_body(pos_hbm, batch_hbm, part_hbm, posb, bb, psum, pcnt, locb):
    wid = _wid()
    iota = lax.iota(jnp.int32, L)
    zeros = jnp.zeros((L,), jnp.float32)
    ones = jnp.full((L,), 1.0, jnp.float32)

    for c in range(3):
        pltpu.sync_copy(pos_hbm.at[pl.ds(c * NPAD + wid * PW, PW)],
                        posb.at[pl.ds(c * PW, PW)])
    pltpu.sync_copy(batch_hbm.at[pl.ds(wid * PW, PW)], bb)

    def zero_body(v, _):
        for l in range(L):
            psum[l, pl.ds(v * L, L)] = zeros
            pcnt[l, pl.ds(v * L, L)] = zeros
        return 0
    lax.fori_loop(0, BINS // L, zero_body, 0)

    def pass1(k, _):
        x = posb[pl.ds(k * L, L)]
        y = posb[pl.ds(PW + k * L, L)]
        z = posb[pl.ds(2 * PW + k * L, L)]
        nrm = _nsqrt(x * x + y * y + z * z)
        b = bb[pl.ds(k * L, L)]
        plsc.addupdate_scatter(psum, [iota, b], nrm)
        plsc.addupdate_scatter(pcnt, [iota, b], ones)
        return 0
    lax.fori_loop(0, KV, pass1, 0)

    def lred(v, _):
        s = psum[0, pl.ds(v * L, L)]
        c = pcnt[0, pl.ds(v * L, L)]
        for l in range(1, L):
            s = s + psum[l, pl.ds(v * L, L)]
            c = c + pcnt[l, pl.ds(v * L, L)]
        locb[pl.ds(v * L, L)] = s
        locb[pl.ds(BINS + v * L, L)] = c
        return 0
    lax.fori_loop(0, BINS // L, lred, 0)

    pltpu.sync_copy(locb, part_hbm.at[wid])


def _norm_body(pos_hbm, batch_hbm, w_hbm, part_hbm, out_hbm,
               posb, outb, bb, partb, scaleb, wv):
    wid = _wid()

    for c in range(3):
        pltpu.sync_copy(pos_hbm.at[pl.ds(c * NPAD + wid * PW, PW)],
                        posb.at[pl.ds(c * PW, PW)])
    pltpu.sync_copy(batch_hbm.at[pl.ds(wid * PW, PW)], bb)
    pltpu.sync_copy(part_hbm, partb)
    pltpu.sync_copy(w_hbm, wv)
    wvec = wv[...]

    def gred(v, _):
        s = partb[0, pl.ds(v * L, L)]
        c = partb[0, pl.ds(BINS + v * L, L)]
        for t in range(1, NW):
            s = s + partb[t, pl.ds(v * L, L)]
            c = c + partb[t, pl.ds(BINS + v * L, L)]
        mean = s / jnp.maximum(c, 1.0)
        scaleb[pl.ds(v * L, L)] = wvec / (mean + EPS)
        return 0
    lax.fori_loop(0, BINS // L, gred, 0)

    def pass2(k, _):
        b = bb[pl.ds(k * L, L)]
        sc = plsc.load_gather(scaleb, [b])
        for c in range(3):
            v = posb[pl.ds(c * PW + k * L, L)]
            outb[pl.ds(c * PW + k * L, L)] = v * sc
        return 0
    lax.fori_loop(0, KV, pass2, 0)

    for c in range(3):
        pltpu.sync_copy(outb.at[pl.ds(c * PW, PW)],
                        out_hbm.at[pl.ds(c * NPAD + wid * PW, PW)])


_stats_sc = pl.kernel(
    _stats_body,
    out_type=jax.ShapeDtypeStruct((NW, 2 * BINS), jnp.float32),
    mesh=_MESH,
    compiler_params=_PARAMS,
    scratch_types=[
        pltpu.VMEM((3 * PW,), jnp.float32),
        pltpu.VMEM((PW,), jnp.int32),
        pltpu.VMEM((L, BINS), jnp.float32),
        pltpu.VMEM((L, BINS), jnp.float32),
        pltpu.VMEM((2 * BINS,), jnp.float32),
    ],
)

_norm_sc = pl.kernel(
    _norm_body,
    out_type=jax.ShapeDtypeStruct((3 * NPAD,), jnp.float32),
    mesh=_MESH,
    compiler_params=_PARAMS,
    scratch_types=[
        pltpu.VMEM((3 * PW,), jnp.float32),
        pltpu.VMEM((3 * PW,), jnp.float32),
        pltpu.VMEM((PW,), jnp.int32),
        pltpu.VMEM((NW, 2 * BINS), jnp.float32),
        pltpu.VMEM((BINS,), jnp.float32),
        pltpu.VMEM((L,), jnp.float32),
    ],
)


def kernel(pos, batch, weight):
    posT = jnp.pad(pos.astype(jnp.float32).T, ((0, 0), (0, NPAD - N)))
    batch_pad = jnp.pad(batch.astype(jnp.int32), (0, NPAD - N),
                        constant_values=NUM_GRAPHS)
    wvec = jnp.full((L,), 1.0, jnp.float32) * weight[0, 0]
    posf = posT.reshape(-1)
    part = _stats_sc(posf, batch_pad)
    outf = _norm_sc(posf, batch_pad, wvec, part)
    return outf.reshape(3, NPAD)[:, :N].T


# speedup vs baseline: 18.0291x; 1.1708x over previous
import jax
import jax.numpy as jnp
from jax import lax
from jax.experimental import pallas as pl
from jax.experimental.pallas import tpu as pltpu
from jax.experimental.pallas import tpu_sc as plsc

N = 100000
NUM_GRAPHS = 256
EPS = 1e-05

L = 16
NC = 2
NS = 16
NW = NC * NS
UNROLL = 4
PW = 3136
NPAD = NW * PW
BINS = 272
KV = PW // (L * UNROLL)

_MESH = plsc.VectorSubcoreMesh(core_axis_name="c", subcore_axis_name="s",
                               num_cores=NC, num_subcores=NS)
_PARAMS = pltpu.CompilerParams(needs_layout_passes=False)


def _norm3(x, y, z):
    q = x * x + y * y + z * z
    i = plsc.bitcast(q, jnp.int32)
    t = plsc.bitcast(0x5F3759DF - (i >> 1), jnp.float32)
    t = t * (1.5 - 0.5 * q * t * t)
    t = t * (1.5 - 0.5 * q * t * t)
    return q * t


def _wid():
    return lax.axis_index("s") * NC + lax.axis_index("c")


def _stats_body(pos_hbm, batch_hbm, part_hbm, posb, bb, psum, pcnt, locb, sem):
    wid = _wid()
    iota = lax.iota(jnp.int32, L)
    zeros = jnp.zeros((L,), jnp.float32)
    ones = jnp.full((L,), 1.0, jnp.float32)

    cps = [pltpu.async_copy(pos_hbm.at[pl.ds(c * NPAD + wid * PW, PW)],
                            posb.at[pl.ds(c * PW, PW)], sem)
           for c in range(3)]
    cps.append(pltpu.async_copy(batch_hbm.at[pl.ds(wid * PW, PW)], bb, sem))

    def zero_body(v, _):
        for l in range(L):
            psum[l, pl.ds(v * L, L)] = zeros
            pcnt[l, pl.ds(v * L, L)] = zeros
        return 0
    lax.fori_loop(0, BINS // L, zero_body, 0)

    for cp in cps:
        cp.wait()

    def pass1(k, _):
        for u in range(UNROLL):
            o = (k * UNROLL + u) * L
            x = posb[pl.ds(o, L)]
            y = posb[pl.ds(PW + o, L)]
            z = posb[pl.ds(2 * PW + o, L)]
            nrm = _norm3(x, y, z)
            b = bb[pl.ds(o, L)]
            plsc.addupdate_scatter(psum, [iota, b], nrm)
            plsc.addupdate_scatter(pcnt, [iota, b], ones)
        return 0
    lax.fori_loop(0, KV, pass1, 0)

    def lred(v, _):
        s = psum[0, pl.ds(v * L, L)]
        c = pcnt[0, pl.ds(v * L, L)]
        for l in range(1, L):
            s = s + psum[l, pl.ds(v * L, L)]
            c = c + pcnt[l, pl.ds(v * L, L)]
        locb[pl.ds(v * L, L)] = s
        locb[pl.ds(BINS + v * L, L)] = c
        return 0
    lax.fori_loop(0, BINS // L, lred, 0)

    pltpu.sync_copy(locb, part_hbm.at[wid])


def _norm_body(pos_hbm, batch_hbm, w_hbm, part_hbm, out_hbm,
               posb, outb, bb, partb, scaleb, wv, sem):
    wid = _wid()

    cps = [pltpu.async_copy(pos_hbm.at[pl.ds(c * NPAD + wid * PW, PW)],
                            posb.at[pl.ds(c * PW, PW)], sem)
           for c in range(3)]
    cps.append(pltpu.async_copy(batch_hbm.at[pl.ds(wid * PW, PW)], bb, sem))
    cps.append(pltpu.async_copy(part_hbm, partb, sem))
    cps.append(pltpu.async_copy(w_hbm, wv, sem))
    for cp in cps:
        cp.wait()
    wvec = wv[...]

    def gred(v, _):
        s = partb[0, pl.ds(v * L, L)]
        c = partb[0, pl.ds(BINS + v * L, L)]
        for t in range(1, NW):
            s = s + partb[t, pl.ds(v * L, L)]
            c = c + partb[t, pl.ds(BINS + v * L, L)]
        mean = s / jnp.maximum(c, 1.0)
        scaleb[pl.ds(v * L, L)] = wvec / (mean + EPS)
        return 0
    lax.fori_loop(0, BINS // L, gred, 0)

    def pass2(k, _):
        for u in range(UNROLL):
            o = (k * UNROLL + u) * L
            b = bb[pl.ds(o, L)]
            sc = plsc.load_gather(scaleb, [b])
            for c in range(3):
                v = posb[pl.ds(c * PW + o, L)]
                outb[pl.ds(c * PW + o, L)] = v * sc
        return 0
    lax.fori_loop(0, KV, pass2, 0)

    for c in range(3):
        pltpu.sync_copy(outb.at[pl.ds(c * PW, PW)],
                        out_hbm.at[pl.ds(c * NPAD + wid * PW, PW)])


_stats_sc = pl.kernel(
    _stats_body,
    out_type=jax.ShapeDtypeStruct((NW, 2 * BINS), jnp.float32),
    mesh=_MESH,
    compiler_params=_PARAMS,
    scratch_types=[
        pltpu.VMEM((3 * PW,), jnp.float32),
        pltpu.VMEM((PW,), jnp.int32),
        pltpu.VMEM((L, BINS), jnp.float32),
        pltpu.VMEM((L, BINS), jnp.float32),
        pltpu.VMEM((2 * BINS,), jnp.float32),
        pltpu.SemaphoreType.DMA,
    ],
)

_norm_sc = pl.kernel(
    _norm_body,
    out_type=jax.ShapeDtypeStruct((3 * NPAD,), jnp.float32),
    mesh=_MESH,
    compiler_params=_PARAMS,
    scratch_types=[
        pltpu.VMEM((3 * PW,), jnp.float32),
        pltpu.VMEM((3 * PW,), jnp.float32),
        pltpu.VMEM((PW,), jnp.int32),
        pltpu.VMEM((NW, 2 * BINS), jnp.float32),
        pltpu.VMEM((BINS,), jnp.float32),
        pltpu.VMEM((L,), jnp.float32),
        pltpu.SemaphoreType.DMA,
    ],
)


def kernel(pos, batch, weight):
    posT = jnp.pad(pos.astype(jnp.float32).T, ((0, 0), (0, NPAD - N)))
    batch_pad = jnp.pad(batch.astype(jnp.int32), (0, NPAD - N),
                        constant_values=NUM_GRAPHS)
    wvec = jnp.full((L,), 1.0, jnp.float32) * weight[0, 0]
    posf = posT.reshape(-1)
    part = _stats_sc(posf, batch_pad)
    outf = _norm_sc(posf, batch_pad, wvec, part)
    return outf.reshape(3, NPAD)[:, :N].T


# speedup vs baseline: 18.6683x; 1.0355x over previous
import jax
import jax.numpy as jnp
from jax import lax
from jax.experimental import pallas as pl
from jax.experimental.pallas import tpu as pltpu
from jax.experimental.pallas import tpu_sc as plsc

N = 100000
NUM_GRAPHS = 256
EPS = 1e-05

L = 16
NC = 2
NS = 16
NW = NC * NS
UNROLL = 8
PW = 3200
NPAD = NW * PW
BINS = 272
KV = PW // (L * UNROLL)

_MESH = plsc.VectorSubcoreMesh(core_axis_name="c", subcore_axis_name="s",
                               num_cores=NC, num_subcores=NS)
_PARAMS = pltpu.CompilerParams(needs_layout_passes=False)


def _norm3(x, y, z):
    q = x * x + y * y + z * z
    i = plsc.bitcast(q, jnp.int32)
    t = plsc.bitcast(0x5F3759DF - (i >> 1), jnp.float32)
    t = t * (1.5 - 0.5 * q * t * t)
    t = t * (1.5 - 0.5 * q * t * t)
    return q * t


def _wid():
    return lax.axis_index("s") * NC + lax.axis_index("c")


def _stats_body(pos_hbm, batch_hbm, part_hbm, posb, bb, psum, pcnt, locb, sem):
    wid = _wid()
    iota = lax.iota(jnp.int32, L)
    zeros = jnp.zeros((L,), jnp.float32)
    ones = jnp.full((L,), 1.0, jnp.float32)

    cps = [pltpu.async_copy(pos_hbm.at[pl.ds(c * NPAD + wid * PW, PW)],
                            posb.at[pl.ds(c * PW, PW)], sem)
           for c in range(3)]
    cps.append(pltpu.async_copy(batch_hbm.at[pl.ds(wid * PW, PW)], bb, sem))

    def zero_body(v, _):
        for l in range(L):
            psum[l, pl.ds(v * L, L)] = zeros
            pcnt[l, pl.ds(v * L, L)] = zeros
        return 0
    lax.fori_loop(0, BINS // L, zero_body, 0)

    for cp in cps:
        cp.wait()

    def pass1(k, _):
        for u in range(UNROLL):
            o = (k * UNROLL + u) * L
            x = posb[pl.ds(o, L)]
            y = posb[pl.ds(PW + o, L)]
            z = posb[pl.ds(2 * PW + o, L)]
            nrm = _norm3(x, y, z)
            b = bb[pl.ds(o, L)]
            plsc.addupdate_scatter(psum, [iota, b], nrm)
            plsc.addupdate_scatter(pcnt, [iota, b], ones)
        return 0
    lax.fori_loop(0, KV, pass1, 0)

    def lred(v, _):
        s = psum[0, pl.ds(v * L, L)]
        c = pcnt[0, pl.ds(v * L, L)]
        for l in range(1, L):
            s = s + psum[l, pl.ds(v * L, L)]
            c = c + pcnt[l, pl.ds(v * L, L)]
        locb[pl.ds(v * L, L)] = s
        locb[pl.ds(BINS + v * L, L)] = c
        return 0
    lax.fori_loop(0, BINS // L, lred, 0)

    pltpu.sync_copy(locb, part_hbm.at[wid])


def _norm_body(pos_hbm, batch_hbm, w_hbm, part_hbm, out_hbm,
               posb, outb, bb, partb, scaleb, wv, sem):
    wid = _wid()

    cps = [pltpu.async_copy(pos_hbm.at[pl.ds(c * NPAD + wid * PW, PW)],
                            posb.at[pl.ds(c * PW, PW)], sem)
           for c in range(3)]
    cps.append(pltpu.async_copy(batch_hbm.at[pl.ds(wid * PW, PW)], bb, sem))
    cps.append(pltpu.async_copy(part_hbm, partb, sem))
    cps.append(pltpu.async_copy(w_hbm, wv, sem))
    for cp in cps:
        cp.wait()
    wvec = wv[...]

    def gred(v, _):
        s = partb[0, pl.ds(v * L, L)]
        c = partb[0, pl.ds(BINS + v * L, L)]
        for t in range(1, NW):
            s = s + partb[t, pl.ds(v * L, L)]
            c = c + partb[t, pl.ds(BINS + v * L, L)]
        mean = s / jnp.maximum(c, 1.0)
        scaleb[pl.ds(v * L, L)] = wvec / (mean + EPS)
        return 0
    lax.fori_loop(0, BINS // L, gred, 0)

    def pass2(k, _):
        for u in range(UNROLL):
            o = (k * UNROLL + u) * L
            b = bb[pl.ds(o, L)]
            sc = plsc.load_gather(scaleb, [b])
            for c in range(3):
                v = posb[pl.ds(c * PW + o, L)]
                outb[pl.ds(c * PW + o, L)] = v * sc
        return 0
    lax.fori_loop(0, KV, pass2, 0)

    for c in range(3):
        pltpu.sync_copy(outb.at[pl.ds(c * PW, PW)],
                        out_hbm.at[pl.ds(c * NPAD + wid * PW, PW)])


_stats_sc = pl.kernel(
    _stats_body,
    out_type=jax.ShapeDtypeStruct((NW, 2 * BINS), jnp.float32),
    mesh=_MESH,
    compiler_params=_PARAMS,
    scratch_types=[
        pltpu.VMEM((3 * PW,), jnp.float32),
        pltpu.VMEM((PW,), jnp.int32),
        pltpu.VMEM((L, BINS), jnp.float32),
        pltpu.VMEM((L, BINS), jnp.float32),
        pltpu.VMEM((2 * BINS,), jnp.float32),
        pltpu.SemaphoreType.DMA,
    ],
)

_norm_sc = pl.kernel(
    _norm_body,
    out_type=jax.ShapeDtypeStruct((3 * NPAD,), jnp.float32),
    mesh=_MESH,
    compiler_params=_PARAMS,
    scratch_types=[
        pltpu.VMEM((3 * PW,), jnp.float32),
        pltpu.VMEM((3 * PW,), jnp.float32),
        pltpu.VMEM((PW,), jnp.int32),
        pltpu.VMEM((NW, 2 * BINS), jnp.float32),
        pltpu.VMEM((BINS,), jnp.float32),
        pltpu.VMEM((L,), jnp.float32),
        pltpu.SemaphoreType.DMA,
    ],
)


def kernel(pos, batch, weight):
    posT = jnp.pad(pos.astype(jnp.float32).T, ((0, 0), (0, NPAD - N)))
    batch_pad = jnp.pad(batch.astype(jnp.int32), (0, NPAD - N),
                        constant_values=NUM_GRAPHS)
    wvec = jnp.full((L,), 1.0, jnp.float32) * weight[0, 0]
    posf = posT.reshape(-1)
    part = _stats_sc(posf, batch_pad)
    outf = _norm_sc(posf, batch_pad, wvec, part)
    return outf.reshape(3, NPAD)[:, :N].T
